# baseline (speedup 1.0000x reference)
import functools

import jax
import jax.numpy as jnp
from jax import lax
from jax.experimental import pallas as pl
from jax.experimental.pallas import tpu as pltpu
from jax.experimental.pallas import tpu_sc as plsc

B = 16384
D = 64
NCOLS = 20
SCALE = 10.0

NW = 32
BPW = B // NW
CHUNK = 64
NCH = BPW // CHUNK
IPC = CHUNK * NCOLS
NIG = IPC // 128
GPC = CHUNK // 16


@functools.partial(
    pl.kernel,
    out_type=jax.ShapeDtypeStruct((B * NCOLS,), jnp.float32),
    mesh=plsc.VectorSubcoreMesh(core_axis_name="c", subcore_axis_name="s"),
    scratch_types=[
        pltpu.VMEM((CHUNK,), jnp.int32),
        pltpu.VMEM((NIG, 128), jnp.int32),
        pltpu.VMEM((CHUNK, D), jnp.float32),
        pltpu.VMEM((IPC, D), jnp.float32),
        pltpu.VMEM((IPC,), jnp.float32),
        pltpu.SemaphoreType.DMA,
        pltpu.SemaphoreType.DMA,
    ],
    compiler_params=pltpu.CompilerParams(
        needs_layout_passes=False, use_tc_tiling_on_sc=False),
)
def _mf_sc(uidx_hbm, iidx_hbm, user_hbm, item_hbm, out_hbm,
           uidx_v, iidx_v, urows_v, irows_v, scores_v, sem_u, sem_i):
    wid = lax.axis_index("s") * 2 + lax.axis_index("c")
    iota16 = lax.iota(jnp.int32, 16)
    iota20 = iota16 * NCOLS
    iota_u = iota16 * D
    iota_i = iota16 * (NCOLS * D)

    def chunk_body(c, carry):
        base = wid * BPW + c * CHUNK
        pltpu.sync_copy(uidx_hbm.at[pl.ds(base, CHUNK)], uidx_v)
        pltpu.sync_copy(iidx_hbm.at[wid * NCH + c], iidx_v)
        cu = pltpu.async_copy(user_hbm.at[uidx_v], urows_v, sem_u)
        cps = [
            pltpu.async_copy(item_hbm.at[iidx_v.at[r]],
                             irows_v.at[pl.ds(r * 128, 128)], sem_i)
            for r in range(NIG)
        ]
        cu.wait()
        for cp in cps:
            cp.wait()

        for g in range(GPC):
            urow = iota16 + g * 16

            def dbody(d, accs):
                dcol = jnp.full((16,), d, jnp.int32)
                uvec = plsc.load_gather(urows_v, [urow, dcol])
                return tuple(
                    acc + uvec * plsc.load_gather(
                        irows_v, [iota20 + (g * 16 * NCOLS + j), dcol])
                    for j, acc in enumerate(accs)
                )

            accs = lax.fori_loop(
                0, D, dbody,
                tuple(jnp.zeros((16,), jnp.float32) for _ in range(NCOLS)))
            for j in range(NCOLS):
                plsc.store_scatter(
                    scores_v, [iota20 + (g * 16 * NCOLS + j)],
                    accs[j] * SCALE)

        pltpu.sync_copy(scores_v, out_hbm.at[pl.ds(base * NCOLS, IPC)])
        return carry

    lax.fori_loop(0, NCH, chunk_body, 0)


def kernel(batch, user_emb, item_emb):
    b = batch.astype(jnp.int32)
    uidx = b[:, 0]
    iidx = b[:, 1:].reshape(NW * NCH, NIG, 128)
    out = _mf_sc(uidx, iidx, user_emb, item_emb)
    return out.reshape(B, NCOLS)


# speedup vs baseline: 1.2972x; 1.2972x over previous
import functools

import jax
import jax.numpy as jnp
from jax import lax
from jax.experimental import pallas as pl
from jax.experimental.pallas import tpu as pltpu
from jax.experimental.pallas import tpu_sc as plsc

B = 16384
D = 64
NCOLS = 20
SCALE = 10.0

NW = 32
BPW = B // NW
CHUNK = 64
NCH = BPW // CHUNK
IPC = CHUNK * NCOLS
NIG = IPC // 128


@functools.partial(
    pl.kernel,
    out_type=jax.ShapeDtypeStruct((B * NCOLS,), jnp.float32),
    mesh=plsc.VectorSubcoreMesh(core_axis_name="c", subcore_axis_name="s"),
    scratch_types=[
        pltpu.VMEM((CHUNK,), jnp.int32),
        pltpu.VMEM((IPC,), jnp.int32),
        pltpu.VMEM((CHUNK, D), jnp.float32),
        pltpu.VMEM((IPC, D), jnp.float32),
        pltpu.VMEM((IPC,), jnp.float32),
        pltpu.SemaphoreType.DMA,
        pltpu.SemaphoreType.DMA,
    ],
    compiler_params=pltpu.CompilerParams(
        needs_layout_passes=False, use_tc_tiling_on_sc=False),
)
def _mf_sc(uidx_hbm, iidx_hbm, user_hbm, item_hbm, out_hbm,
           uidx_v, iidx_v, urows_v, irows_v, scores_v, sem_u, sem_i):
    wid = lax.axis_index("s") * 2 + lax.axis_index("c")
    iota16 = lax.iota(jnp.int32, 16)

    def chunk_body(c, carry):
        base = wid * BPW + c * CHUNK
        pltpu.sync_copy(uidx_hbm.at[pl.ds(base, CHUNK)], uidx_v)
        pltpu.sync_copy(iidx_hbm.at[pl.ds(base * NCOLS, IPC)], iidx_v)
        cu = pltpu.async_copy(user_hbm.at[uidx_v], urows_v, sem_u)
        cps = [
            pltpu.async_copy(item_hbm.at[iidx_v.at[pl.ds(r * 128, 128)]],
                             irows_v.at[pl.ds(r * 128, 128)], sem_i)
            for r in range(NIG)
        ]
        cu.wait()
        for cp in cps:
            cp.wait()

        def bbody(b4, inner):
            res = [jnp.zeros((16,), jnp.float32) for _ in range(5)]
            for bb in range(4):
                b = b4 * 4 + bb
                u = [urows_v[b, pl.ds(k * 16, 16)] for k in range(D // 16)]
                for j in range(NCOLS):
                    row = b * NCOLS + j
                    prod = u[0] * irows_v[row, pl.ds(0, 16)]
                    for k in range(1, D // 16):
                        prod = prod + u[k] * irows_v[row, pl.ds(k * 16, 16)]
                    s = jnp.sum(prod)
                    o = bb * NCOLS + j
                    res[o // 16] = jnp.where(iota16 == (o % 16), s,
                                             res[o // 16])
            for v in range(5):
                scores_v[pl.ds(b4 * 80 + v * 16, 16)] = res[v] * SCALE
            return inner

        lax.fori_loop(0, CHUNK // 4, bbody, 0)
        pltpu.sync_copy(scores_v, out_hbm.at[pl.ds(base * NCOLS, IPC)])
        return carry

    lax.fori_loop(0, NCH, chunk_body, 0)


def kernel(batch, user_emb, item_emb):
    b = batch.astype(jnp.int32)
    uidx = b[:, 0]
    iidx = b[:, 1:].reshape(B * NCOLS)
    out = _mf_sc(uidx, iidx, user_emb, item_emb)
    return out.reshape(B, NCOLS)
